# baseline (speedup 1.0000x reference)
import functools

import jax
import jax.numpy as jnp
from jax import lax
from jax.experimental import pallas as pl
from jax.experimental.pallas import tpu as pltpu
from jax.experimental.pallas import tpu_sc as plsc


def _sc_delta_kernel(n_ids: int):
    info = plsc.get_sparse_core_info()
    nc, ns, lanes = info.num_cores, info.num_subcores, info.num_lanes
    nw = nc * ns
    per_w = n_ids // nw
    assert n_ids % (8 * nw) == 0
    chunk = 128
    assert per_w % chunk == 0

    mesh = plsc.VectorSubcoreMesh(core_axis_name="c", subcore_axis_name="s")

    @functools.partial(
        pl.kernel,
        mesh=mesh,
        out_type=jax.ShapeDtypeStruct((n_ids,), jnp.float32),
        scratch_types=[
            pltpu.VMEM((per_w,), jnp.int32),
            pltpu.VMEM((per_w,), jnp.float32),
            pltpu.VMEM((per_w,), jnp.float32),
            pltpu.SemaphoreType.DMA,
        ],
    )
    def sc_delta(ids_hbm, table_hbm, out_hbm, idx_v, rows_v, delta_v, sem):
        wid = lax.axis_index("s") * nc + lax.axis_index("c")
        base = wid * per_w
        pltpu.sync_copy(ids_hbm.at[pl.ds(base, per_w)], idx_v)
        copies = [
            pltpu.async_copy(
                table_hbm.at[idx_v.at[pl.ds(c * chunk, chunk)]],
                rows_v.at[pl.ds(c * chunk, chunk)],
                sem,
            )
            for c in range(per_w // chunk)
        ]
        for cop in copies:
            cop.wait()
        for i in range(per_w // lanes):
            sl = pl.ds(i * lanes, lanes)
            v = rows_v[sl]
            delta_v[sl] = v - v
        pltpu.sync_copy(delta_v, out_hbm.at[pl.ds(base, per_w)])

    return sc_delta


def _tc_add_kernel(seq: int, batch: int, d_model: int, bs: int):
    def body(h_ref, d_ref, o_ref):
        o_ref[...] = h_ref[...] + d_ref[...][:, :, None]

    return pl.pallas_call(
        body,
        grid=(seq // bs,),
        in_specs=[
            pl.BlockSpec((bs, batch, d_model), lambda i: (i, 0, 0)),
            pl.BlockSpec((bs, batch), lambda i: (i, 0)),
        ],
        out_specs=pl.BlockSpec((bs, batch, d_model), lambda i: (i, 0, 0)),
        out_shape=jax.ShapeDtypeStruct((seq, batch, d_model), jnp.float32),
    )


@functools.lru_cache(maxsize=None)
def _build(batch, seq, d_model):
    return _sc_delta_kernel(batch * seq), _tc_add_kernel(seq, batch, d_model, 256)


def kernel(witness_ids, hidden_states, witness_weight):
    batch, seq = witness_ids.shape
    seq_h, batch_h, d_model = hidden_states.shape
    sc_delta, tc_add = _build(batch, seq, d_model)
    ids_sb = witness_ids.T.reshape(-1).astype(jnp.int32)
    table = witness_weight.reshape(-1)
    delta = sc_delta(ids_sb, table)
    return tc_add(hidden_states, delta.reshape(seq, batch))
